# baseline (speedup 1.0000x reference)
import functools

import jax
import jax.numpy as jnp
from jax.experimental import pallas as pl


def _gather_kernel(ids_ref, table_ref, out_ref):
    ids = ids_ref[0]
    v = table_ref.shape[0]
    iota = jax.lax.broadcasted_iota(jnp.int32, (ids.shape[0], v), 1)
    onehot = (ids == iota).astype(jnp.float32)
    out_ref[...] = jnp.dot(onehot, table_ref[...],
                           preferred_element_type=jnp.float32
                           ).astype(jnp.bfloat16)


def _conv_highway_kernel(x_ref, wstack_ref, cb_ref, wp_ref, bp_ref,
                         wg_ref, bg_ref, out_ref, *, kk, tt):
    nb, maxw, e = x_ref.shape
    xm = x_ref[...].reshape(nb * maxw, e)
    acc = jnp.zeros((nb, tt, e), jnp.float32)
    for k in range(kk):
        q = jnp.dot(xm, wstack_ref[k],
                    preferred_element_type=jnp.float32).reshape(nb, maxw, e)
        acc = acc + q[:, k:k + tt, :]
    conv = jnp.maximum(acc + cb_ref[0][None, None, :], 0.0)
    cnn = jnp.max(conv, axis=1)
    proj = jnp.maximum(
        jnp.dot(cnn, wp_ref[...], preferred_element_type=jnp.float32)
        + bp_ref[0][None, :], 0.0)
    gate = jax.nn.sigmoid(
        jnp.dot(cnn, wg_ref[...], preferred_element_type=jnp.float32)
        + bg_ref[0][None, :])
    out_ref[...] = gate * proj + (1.0 - gate) * cnn


def kernel(input_tensor, emb_table, conv_w, conv_b, w_proj, b_proj,
           w_gate, b_gate):
    s, b, maxw = input_tensor.shape
    v, e = emb_table.shape
    kk = conv_w.shape[2]
    n = s * b
    tt = maxw - kk + 1

    ids = input_tensor.astype(jnp.int32).reshape(-1)
    rows = ids.shape[0]
    gblocks = 32
    r = rows // gblocks
    ids3 = ids.reshape(gblocks, r, 1)

    y = pl.pallas_call(
        _gather_kernel,
        grid=(gblocks,),
        in_specs=[
            pl.BlockSpec((1, r, 1), lambda i: (i, 0, 0)),
            pl.BlockSpec((v, e), lambda i: (0, 0)),
        ],
        out_specs=pl.BlockSpec((r, e), lambda i: (i, 0)),
        out_shape=jax.ShapeDtypeStruct((rows, e), jnp.bfloat16),
    )(ids3, emb_table)

    p = y.reshape(n, e, maxw).transpose(0, 2, 1)

    wstack = conv_w.transpose(2, 1, 0).astype(jnp.bfloat16)
    cb2 = conv_b.reshape(1, e)
    bp2 = b_proj.reshape(1, e)
    bg2 = b_gate.reshape(1, e)
    wpt = w_proj.T
    wgt = w_gate.T

    nb = 256
    nblocks = n // nb
    out = pl.pallas_call(
        functools.partial(_conv_highway_kernel, kk=kk, tt=tt),
        grid=(nblocks,),
        in_specs=[
            pl.BlockSpec((nb, maxw, e), lambda i: (i, 0, 0)),
            pl.BlockSpec((kk, e, e), lambda i: (0, 0, 0)),
            pl.BlockSpec((1, e), lambda i: (0, 0)),
            pl.BlockSpec((e, e), lambda i: (0, 0)),
            pl.BlockSpec((1, e), lambda i: (0, 0)),
            pl.BlockSpec((e, e), lambda i: (0, 0)),
            pl.BlockSpec((1, e), lambda i: (0, 0)),
        ],
        out_specs=pl.BlockSpec((nb, e), lambda i: (i, 0)),
        out_shape=jax.ShapeDtypeStruct((n, e), jnp.float32),
    )(p, wstack, cb2, wpt, bp2, wgt, bg2)

    return out.reshape(s, b, e)


# speedup vs baseline: 2.8856x; 2.8856x over previous
import functools

import jax
import jax.numpy as jnp
from jax.experimental import pallas as pl


def _gather_kernel(ids_ref, table_ref, out_ref):
    ids = ids_ref[0, 0]
    v = table_ref.shape[0]
    iota = jax.lax.broadcasted_iota(jnp.int32, (ids.shape[0], v), 1)
    onehot = (ids == iota).astype(jnp.bfloat16)
    out_ref[...] = jnp.dot(onehot, table_ref[...],
                           preferred_element_type=jnp.float32
                           ).astype(jnp.bfloat16)


def _conv_highway_kernel(y_ref, wpad_ref, cb_ref, wp_ref, bp_ref,
                         wg_ref, bg_ref, out_ref, *, tt):
    y = y_ref[...]
    je = y.shape[1]
    pt = tt - 1
    cb = cb_ref[0][None, :]
    mx = None
    for t in range(tt):
        wt = wpad_ref[pl.ds(pt - t, je), :]
        acc = jnp.dot(y, wt, preferred_element_type=jnp.float32)
        z = jnp.maximum(acc + cb, 0.0)
        mx = z if mx is None else jnp.maximum(mx, z)
    cnn = mx
    proj = jnp.maximum(
        jnp.dot(cnn, wp_ref[...], preferred_element_type=jnp.float32)
        + bp_ref[0][None, :], 0.0)
    gate = jax.nn.sigmoid(
        jnp.dot(cnn, wg_ref[...], preferred_element_type=jnp.float32)
        + bg_ref[0][None, :])
    out_ref[...] = gate * proj + (1.0 - gate) * cnn


def kernel(input_tensor, emb_table, conv_w, conv_b, w_proj, b_proj,
           w_gate, b_gate):
    s, b, maxw = input_tensor.shape
    v, e = emb_table.shape
    kk = conv_w.shape[2]
    n = s * b
    tt = maxw - kk + 1
    pt = tt - 1
    je = maxw * e

    nbw = 512
    gwb = n // nbw
    ids_w = (input_tensor.astype(jnp.int32).reshape(n, maxw).T
             .reshape(maxw, gwb, nbw, 1))
    table_b = emb_table.astype(jnp.bfloat16)

    y = pl.pallas_call(
        _gather_kernel,
        grid=(gwb, maxw),
        in_specs=[
            pl.BlockSpec((1, 1, nbw, 1), lambda i, w: (w, i, 0, 0)),
            pl.BlockSpec((v, e), lambda i, w: (0, 0)),
        ],
        out_specs=pl.BlockSpec((nbw, e), lambda i, w: (i, w)),
        out_shape=jax.ShapeDtypeStruct((n, je), jnp.bfloat16),
    )(ids_w, table_b)

    wlin = (jnp.pad(conv_w, ((0, 0), (0, 0), (0, maxw - kk)))
            .transpose(1, 2, 0).reshape(je, e))
    wpad = jnp.pad(wlin, ((pt, 0), (0, 0))).astype(jnp.bfloat16)

    cb2 = conv_b.reshape(1, e)
    bp2 = b_proj.reshape(1, e)
    bg2 = b_gate.reshape(1, e)
    wpt = w_proj.T
    wgt = w_gate.T

    nb = 1024
    nblocks = n // nb
    out = pl.pallas_call(
        functools.partial(_conv_highway_kernel, tt=tt),
        grid=(nblocks,),
        in_specs=[
            pl.BlockSpec((nb, je), lambda i: (i, 0)),
            pl.BlockSpec((pt + je, e), lambda i: (0, 0)),
            pl.BlockSpec((1, e), lambda i: (0, 0)),
            pl.BlockSpec((e, e), lambda i: (0, 0)),
            pl.BlockSpec((1, e), lambda i: (0, 0)),
            pl.BlockSpec((e, e), lambda i: (0, 0)),
            pl.BlockSpec((1, e), lambda i: (0, 0)),
        ],
        out_specs=pl.BlockSpec((nb, e), lambda i: (i, 0)),
        out_shape=jax.ShapeDtypeStruct((n, e), jnp.float32),
    )(y, wpad, cb2, wpt, bp2, wgt, bg2)

    return out.reshape(s, b, e)


# speedup vs baseline: 3.2523x; 1.1271x over previous
import functools

import jax
import jax.numpy as jnp
from jax.experimental import pallas as pl


def _gather_kernel(ids_ref, table_ref, out_ref):
    ids = ids_ref[0, 0]
    v = table_ref.shape[0]
    iota = jax.lax.broadcasted_iota(jnp.int32, (ids.shape[0], v), 1)
    onehot = (ids == iota).astype(jnp.bfloat16)
    out_ref[...] = jnp.dot(onehot, table_ref[...],
                           preferred_element_type=jnp.float32
                           ).astype(jnp.bfloat16)


def _conv_highway_kernel(y_ref, wpad_ref, cb_ref, wp_ref, bp_ref,
                         wg_ref, bg_ref, out_ref, *, tt):
    y = y_ref[...]
    je = y.shape[1]
    pt = tt - 1
    cb = cb_ref[0][None, :]
    mx = None
    for t in range(tt):
        wt = wpad_ref[pl.ds(pt - t, je), :]
        acc = jnp.dot(y, wt, preferred_element_type=jnp.float32)
        z = jnp.maximum(acc + cb, 0.0)
        mx = z if mx is None else jnp.maximum(mx, z)
    cnn = mx
    proj = jnp.maximum(
        jnp.dot(cnn, wp_ref[...], preferred_element_type=jnp.float32)
        + bp_ref[0][None, :], 0.0)
    gate = jax.nn.sigmoid(
        jnp.dot(cnn, wg_ref[...], preferred_element_type=jnp.float32)
        + bg_ref[0][None, :])
    out_ref[...] = gate * proj + (1.0 - gate) * cnn


def kernel(input_tensor, emb_table, conv_w, conv_b, w_proj, b_proj,
           w_gate, b_gate):
    s, b, maxw = input_tensor.shape
    v, e = emb_table.shape
    kk = conv_w.shape[2]
    n = s * b
    tt = maxw - kk + 1
    pt = tt - 1
    je = maxw * e

    nbw = 1024
    gwb = n // nbw
    ids_w = (input_tensor.astype(jnp.int32).reshape(n, maxw).T
             .reshape(maxw, gwb, nbw, 1))
    table_b = emb_table.astype(jnp.bfloat16)

    y = pl.pallas_call(
        _gather_kernel,
        grid=(gwb, maxw),
        in_specs=[
            pl.BlockSpec((1, 1, nbw, 1), lambda i, w: (w, i, 0, 0)),
            pl.BlockSpec((v, e), lambda i, w: (0, 0)),
        ],
        out_specs=pl.BlockSpec((nbw, e), lambda i, w: (i, w)),
        out_shape=jax.ShapeDtypeStruct((n, je), jnp.bfloat16),
    )(ids_w, table_b)

    wlin = (jnp.pad(conv_w, ((0, 0), (0, 0), (0, maxw - kk)))
            .transpose(1, 2, 0).reshape(je, e))
    wpad = jnp.pad(wlin, ((pt, 0), (0, 0))).astype(jnp.bfloat16)

    cb2 = conv_b.reshape(1, e)
    bp2 = b_proj.reshape(1, e)
    bg2 = b_gate.reshape(1, e)
    wpt = w_proj.T
    wgt = w_gate.T

    nb = 1024
    nblocks = n // nb
    out = pl.pallas_call(
        functools.partial(_conv_highway_kernel, tt=tt),
        grid=(nblocks,),
        in_specs=[
            pl.BlockSpec((nb, je), lambda i: (i, 0)),
            pl.BlockSpec((pt + je, e), lambda i: (0, 0)),
            pl.BlockSpec((1, e), lambda i: (0, 0)),
            pl.BlockSpec((e, e), lambda i: (0, 0)),
            pl.BlockSpec((1, e), lambda i: (0, 0)),
            pl.BlockSpec((e, e), lambda i: (0, 0)),
            pl.BlockSpec((1, e), lambda i: (0, 0)),
        ],
        out_specs=pl.BlockSpec((nb, e), lambda i: (i, 0)),
        out_shape=jax.ShapeDtypeStruct((n, e), jnp.float32),
    )(y, wpad, cb2, wpt, bp2, wgt, bg2)

    return out.reshape(s, b, e)


# speedup vs baseline: 3.3861x; 1.0412x over previous
import functools

import jax
import jax.numpy as jnp
from jax.experimental import pallas as pl
from jax.experimental.pallas import tpu as pltpu


def _fused_kernel(ids_ref, table_ref, wpad_ref, cb_ref, wp_ref, bp_ref,
                  wg_ref, bg_ref, out_ref, y_scr, *, tt, maxw):
    nb = out_ref.shape[0]
    v, e = table_ref.shape
    w = pl.program_id(1)

    @pl.when(w < maxw)
    def _gather():
        idc = ids_ref[0, 0]
        iota = jax.lax.broadcasted_iota(jnp.int32, (nb, v), 1)
        onehot = (idc == iota).astype(jnp.bfloat16)
        g = jnp.dot(onehot, table_ref[...],
                    preferred_element_type=jnp.float32).astype(jnp.bfloat16)
        y_scr[:, pl.ds(w * e, e)] = g

    @pl.when(w == maxw)
    def _conv_highway():
        je = maxw * e
        pt = tt - 1
        cb = cb_ref[0][None, :]
        mx = None
        for t in range(tt):
            wt = wpad_ref[pl.ds(pt - t, je), :]
            acc = jnp.dot(y_scr[...], wt, preferred_element_type=jnp.float32)
            z = jnp.maximum(acc + cb, 0.0)
            mx = z if mx is None else jnp.maximum(mx, z)
        cnn = mx
        proj = jnp.maximum(
            jnp.dot(cnn, wp_ref[...], preferred_element_type=jnp.float32)
            + bp_ref[0][None, :], 0.0)
        gate = jax.nn.sigmoid(
            jnp.dot(cnn, wg_ref[...], preferred_element_type=jnp.float32)
            + bg_ref[0][None, :])
        out_ref[...] = gate * proj + (1.0 - gate) * cnn


def kernel(input_tensor, emb_table, conv_w, conv_b, w_proj, b_proj,
           w_gate, b_gate):
    s, b, maxw = input_tensor.shape
    v, e = emb_table.shape
    kk = conv_w.shape[2]
    n = s * b
    tt = maxw - kk + 1
    pt = tt - 1
    je = maxw * e

    nb = 2048
    nblocks = n // nb
    ids4 = (input_tensor.astype(jnp.int32).reshape(nblocks, nb, maxw)
            .transpose(0, 2, 1)[..., None])
    table_b = emb_table.astype(jnp.bfloat16)

    wlin = (jnp.pad(conv_w, ((0, 0), (0, 0), (0, maxw - kk)))
            .transpose(1, 2, 0).reshape(je, e))
    wpad = jnp.pad(wlin, ((pt, 0), (0, 0))).astype(jnp.bfloat16)

    cb2 = conv_b.reshape(1, e)
    bp2 = b_proj.reshape(1, e)
    bg2 = b_gate.reshape(1, e)
    wpt = w_proj.T
    wgt = w_gate.T

    mwc = maxw - 1
    out = pl.pallas_call(
        functools.partial(_fused_kernel, tt=tt, maxw=maxw),
        grid=(nblocks, maxw + 1),
        in_specs=[
            pl.BlockSpec((1, 1, nb, 1),
                         lambda i, w: (i, jnp.minimum(w, mwc), 0, 0)),
            pl.BlockSpec((v, e), lambda i, w: (0, 0)),
            pl.BlockSpec((pt + je, e), lambda i, w: (0, 0)),
            pl.BlockSpec((1, e), lambda i, w: (0, 0)),
            pl.BlockSpec((e, e), lambda i, w: (0, 0)),
            pl.BlockSpec((1, e), lambda i, w: (0, 0)),
            pl.BlockSpec((e, e), lambda i, w: (0, 0)),
            pl.BlockSpec((1, e), lambda i, w: (0, 0)),
        ],
        out_specs=pl.BlockSpec((nb, e), lambda i, w: (i, 0)),
        out_shape=jax.ShapeDtypeStruct((n, e), jnp.float32),
        scratch_shapes=[pltpu.VMEM((nb, je), jnp.bfloat16)],
    )(ids4, table_b, wpad, cb2, wpt, bp2, wgt, bg2)

    return out.reshape(s, b, e)


# speedup vs baseline: 3.6126x; 1.0669x over previous
import functools

import jax
import jax.numpy as jnp
from jax.experimental import pallas as pl
from jax.experimental.pallas import tpu as pltpu


def _fused_kernel(ids_ref, table_ref, wpad_ref, cb_ref, wp_ref, bp_ref,
                  wg_ref, bg_ref, out_ref, y_scr, *, tt, maxw, cps, wsteps):
    nb = out_ref.shape[0]
    v, e = table_ref.shape
    w = pl.program_id(1)

    @pl.when(w < wsteps)
    def _gather():
        iota = jax.lax.broadcasted_iota(jnp.int32, (nb, v), 1)
        table = table_ref[...]
        for j in range(cps):
            idc = ids_ref[0, j]
            onehot = (idc == iota).astype(jnp.bfloat16)
            g = jnp.dot(onehot, table,
                        preferred_element_type=jnp.float32
                        ).astype(jnp.bfloat16)
            y_scr[:, pl.ds((w * cps + j) * e, e)] = g

    @pl.when(w == wsteps)
    def _conv_highway():
        je = maxw * e
        pt = tt - 1
        cb = cb_ref[0][None, :]
        mx = None
        for t in range(tt):
            wt = wpad_ref[pl.ds(pt - t, je), :]
            acc = jnp.dot(y_scr[...], wt, preferred_element_type=jnp.float32)
            z = jnp.maximum(acc + cb, 0.0)
            mx = z if mx is None else jnp.maximum(mx, z)
        cnn = mx
        proj = jnp.maximum(
            jnp.dot(cnn, wp_ref[...], preferred_element_type=jnp.float32)
            + bp_ref[0][None, :], 0.0)
        gate = jax.nn.sigmoid(
            jnp.dot(cnn, wg_ref[...], preferred_element_type=jnp.float32)
            + bg_ref[0][None, :])
        out_ref[...] = gate * proj + (1.0 - gate) * cnn


def kernel(input_tensor, emb_table, conv_w, conv_b, w_proj, b_proj,
           w_gate, b_gate):
    s, b, maxw = input_tensor.shape
    v, e = emb_table.shape
    kk = conv_w.shape[2]
    n = s * b
    tt = maxw - kk + 1
    pt = tt - 1
    je = maxw * e

    nb = 2048
    nblocks = n // nb
    ids4 = (input_tensor.astype(jnp.int32).reshape(nblocks, nb, maxw)
            .transpose(0, 2, 1)[..., None])
    table_b = emb_table.astype(jnp.bfloat16)

    wlin = (jnp.pad(conv_w, ((0, 0), (0, 0), (0, maxw - kk)))
            .transpose(1, 2, 0).reshape(je, e))
    wpad = jnp.pad(wlin, ((pt, 0), (0, 0))).astype(jnp.bfloat16)

    cb2 = conv_b.reshape(1, e)
    bp2 = b_proj.reshape(1, e)
    bg2 = b_gate.reshape(1, e)
    wpt = w_proj.T
    wgt = w_gate.T

    cps = 3
    wsteps = maxw // cps
    wlast = wsteps - 1
    out = pl.pallas_call(
        functools.partial(_fused_kernel, tt=tt, maxw=maxw, cps=cps,
                          wsteps=wsteps),
        grid=(nblocks, wsteps + 1),
        in_specs=[
            pl.BlockSpec((1, cps, nb, 1),
                         lambda i, w: (i, jnp.minimum(w, wlast), 0, 0)),
            pl.BlockSpec((v, e), lambda i, w: (0, 0)),
            pl.BlockSpec((pt + je, e), lambda i, w: (0, 0)),
            pl.BlockSpec((1, e), lambda i, w: (0, 0)),
            pl.BlockSpec((e, e), lambda i, w: (0, 0)),
            pl.BlockSpec((1, e), lambda i, w: (0, 0)),
            pl.BlockSpec((e, e), lambda i, w: (0, 0)),
            pl.BlockSpec((1, e), lambda i, w: (0, 0)),
        ],
        out_specs=pl.BlockSpec((nb, e), lambda i, w: (i, 0)),
        out_shape=jax.ShapeDtypeStruct((n, e), jnp.float32),
        scratch_shapes=[pltpu.VMEM((nb, je), jnp.bfloat16)],
    )(ids4, table_b, wpad, cb2, wpt, bp2, wgt, bg2)

    return out.reshape(s, b, e)


# speedup vs baseline: 4.1143x; 1.1389x over previous
import functools

import jax
import jax.numpy as jnp
from jax.experimental import pallas as pl
from jax.experimental.pallas import tpu as pltpu


def _fused_kernel(ids_ref, table_ref, wpad_ref, cb_ref, wp_ref, bp_ref,
                  wg_ref, bg_ref, out_ref, y_scr, *, tt, maxw, cps, wsteps):
    nb = out_ref.shape[0]
    v, e = table_ref.shape
    w = pl.program_id(1)

    @pl.when(w < wsteps)
    def _gather():
        iota = jax.lax.broadcasted_iota(jnp.int32, (nb, v), 1)
        table = table_ref[...]
        for j in range(cps):
            idc = ids_ref[0, j]
            onehot = (idc == iota).astype(jnp.bfloat16)
            g = jnp.dot(onehot, table,
                        preferred_element_type=jnp.float32
                        ).astype(jnp.bfloat16)
            y_scr[:, pl.ds((w * cps + j) * e, e)] = g

    @pl.when(w == wsteps)
    def _conv_highway():
        je = maxw * e
        pt = tt - 1
        cb = cb_ref[0][None, :]
        mx = None
        for t in range(tt):
            wt = wpad_ref[pl.ds(pt - t, je), :]
            acc = jnp.dot(y_scr[...], wt, preferred_element_type=jnp.float32)
            z = jnp.maximum(acc + cb, 0.0)
            mx = z if mx is None else jnp.maximum(mx, z)
        cnn = mx
        proj = jnp.maximum(
            jnp.dot(cnn, wp_ref[...], preferred_element_type=jnp.float32)
            + bp_ref[0][None, :], 0.0)
        gate = jax.nn.sigmoid(
            jnp.dot(cnn, wg_ref[...], preferred_element_type=jnp.float32)
            + bg_ref[0][None, :])
        out_ref[...] = gate * proj + (1.0 - gate) * cnn


def kernel(input_tensor, emb_table, conv_w, conv_b, w_proj, b_proj,
           w_gate, b_gate):
    s, b, maxw = input_tensor.shape
    v, e = emb_table.shape
    kk = conv_w.shape[2]
    n = s * b
    tt = maxw - kk + 1
    pt = tt - 1
    je = maxw * e

    nb = 2048
    nblocks = n // nb
    ids4 = (input_tensor.astype(jnp.int32).reshape(nblocks, nb, maxw)
            .transpose(0, 2, 1)[..., None])
    table_b = emb_table.astype(jnp.bfloat16)

    wlin = (jnp.pad(conv_w, ((0, 0), (0, 0), (0, maxw - kk)))
            .transpose(1, 2, 0).reshape(je, e))
    wpad = jnp.pad(wlin, ((pt, 0), (0, 0))).astype(jnp.bfloat16)

    cb2 = conv_b.reshape(1, e)
    bp2 = b_proj.reshape(1, e)
    bg2 = b_gate.reshape(1, e)
    wpt = w_proj.T
    wgt = w_gate.T

    cps = 7
    wsteps = maxw // cps
    wlast = wsteps - 1
    out = pl.pallas_call(
        functools.partial(_fused_kernel, tt=tt, maxw=maxw, cps=cps,
                          wsteps=wsteps),
        grid=(nblocks, wsteps + 1),
        in_specs=[
            pl.BlockSpec((1, cps, nb, 1),
                         lambda i, w: (i, jnp.minimum(w, wlast), 0, 0)),
            pl.BlockSpec((v, e), lambda i, w: (0, 0)),
            pl.BlockSpec((pt + je, e), lambda i, w: (0, 0)),
            pl.BlockSpec((1, e), lambda i, w: (0, 0)),
            pl.BlockSpec((e, e), lambda i, w: (0, 0)),
            pl.BlockSpec((1, e), lambda i, w: (0, 0)),
            pl.BlockSpec((e, e), lambda i, w: (0, 0)),
            pl.BlockSpec((1, e), lambda i, w: (0, 0)),
        ],
        out_specs=pl.BlockSpec((nb, e), lambda i, w: (i, 0)),
        out_shape=jax.ShapeDtypeStruct((n, e), jnp.float32),
        scratch_shapes=[pltpu.VMEM((nb, je), jnp.bfloat16)],
    )(ids4, table_b, wpad, cb2, wpt, bp2, wgt, bg2)

    return out.reshape(s, b, e)
